# Initial kernel scaffold; baseline (speedup 1.0000x reference)
import jax
import jax.numpy as jnp
from jax.experimental import pallas as pl


def kernel(x, edge_index, d_w, d_b, f_w, f_b, W, b_w, a_w, a_b):
    raise NotImplementedError("write your pallas kernel here")


# speedup vs baseline: 16.3254x; 16.3254x over previous
import functools
import jax
import jax.numpy as jnp
from jax import lax
from jax.experimental import pallas as pl
from jax.experimental.pallas import tpu as pltpu
from jax.experimental.pallas import tpu_sc as plsc

N = 10000
E = 320000
D = 128
HEAD = 4
HD = 32
NP = 10240
NC = 2
NS = 16
NW = NC * NS
EPW = E // NW
CH = 80
NCHUNK = EPW // CH
ROWS_PER_TILE = N // NS
ACCW = 144
ACCN = N + 16


def _prep_body(x_ref, w_ref, bw_ref, g_ref, tb_ref, h_ref, t_ref):
    xb = x_ref[...].astype(jnp.bfloat16)
    wb = w_ref[...].astype(jnp.bfloat16)
    h = jnp.dot(xb, wb, preferred_element_type=jnp.float32) + bw_ref[...]
    h_ref[...] = h
    hb = h.astype(jnp.bfloat16)
    gb = g_ref[...].astype(jnp.bfloat16)
    t_ref[...] = jnp.dot(hb, gb,
                         preferred_element_type=jnp.float32) + tb_ref[...]


def _prep(xp, w, bw, g, tb):
    return pl.pallas_call(
        _prep_body,
        grid=(NP // 256,),
        in_specs=[
            pl.BlockSpec((256, 128), lambda i: (i, 0)),
            pl.BlockSpec((128, 128), lambda i: (0, 0)),
            pl.BlockSpec((1, 128), lambda i: (0, 0)),
            pl.BlockSpec((128, 16), lambda i: (0, 0)),
            pl.BlockSpec((1, 16), lambda i: (0, 0)),
        ],
        out_specs=[
            pl.BlockSpec((256, 128), lambda i: (i, 0)),
            pl.BlockSpec((256, 16), lambda i: (i, 0)),
        ],
        out_shape=[
            jax.ShapeDtypeStruct((NP, 128), jnp.float32),
            jax.ShapeDtypeStruct((NP, 16), jnp.float32),
        ],
    )(xp, w, bw, g, tb)


def _edge_body(src_hbm, dst_hbm, sgn_hbm, t_hbm, h_hbm, num_hbm, den_hbm,
               acc, idx_s, idx_d, sgbuf, bufs, bufd, bufh, srow, wbuf,
               postab, winbuf, lbuf, sem):
    cid = lax.axis_index("c")
    sid = lax.axis_index("s")
    wid = cid * NS + sid
    zero16 = jnp.zeros((16,), jnp.float32)

    def _zrow(r, _):
        for j in range(ACCW // 16):
            srow[r, pl.ds(j * 16, 16)] = zero16
        return 0
    lax.fori_loop(0, CH, _zrow, 0)
    row0 = sid * ROWS_PER_TILE
    for tch in range(ROWS_PER_TILE // CH):
        pltpu.sync_copy(srow, acc.at[pl.ds(row0 + tch * CH, CH)])
    rem = ROWS_PER_TILE % CH
    if rem:
        pltpu.sync_copy(srow.at[pl.ds(0, rem)],
                        acc.at[pl.ds(row0 + (ROWS_PER_TILE // CH) * CH, rem)])
    @pl.when(sid == 0)
    def _zero_spare():
        pltpu.sync_copy(srow.at[pl.ds(0, ACCN - N)], acc.at[pl.ds(N, ACCN - N)])
    plsc.subcore_barrier()

    lanes = lax.iota(jnp.int32, 16)
    ebase = wid * EPW

    def _chunk(c, _):
        off = ebase + c * CH
        pltpu.sync_copy(src_hbm.at[pl.ds(off, CH)], idx_s)
        pltpu.sync_copy(dst_hbm.at[pl.ds(off, CH)], idx_d)
        pltpu.sync_copy(sgn_hbm.at[pl.ds(off, CH)], sgbuf)
        c1 = pltpu.async_copy(t_hbm.at[idx_s], bufs, sem)
        c2 = pltpu.async_copy(t_hbm.at[idx_d], bufd, sem)
        c3 = pltpu.async_copy(h_hbm.at[idx_s], bufh, sem)
        c1.wait()
        c2.wait()
        c3.wait()
        for g in range(CH // 16):
            rows = g * 16 + lanes
            col = lambda j: jnp.full((16,), j, jnp.int32)
            sgn = sgbuf[pl.ds(g * 16, 16)]
            for hh in range(HEAD):
                ps = plsc.load_gather(bufs, [rows, col(hh)])
                qd = plsc.load_gather(bufd, [rows, col(4 + hh)])
                t = sgn * ps + qd
                al = jnp.where(t >= 0.0, t, 0.01 * t)
                ex = jnp.exp(al)
                plsc.store_scatter(srow, [rows, col(128 + hh)], ex)
                wbuf[hh, pl.ds(g * 16, 16)] = ex * sgn
        for g in range(CH // 16):
            wv = [wbuf[hh, pl.ds(g * 16, 16)] for hh in range(HEAD)]
            for i in range(16):
                e = g * 16 + i
                for k in range(8):
                    srow[e, pl.ds(k * 16, 16)] = (
                        bufh[e, pl.ds(k * 16, 16)] * wv[k // 2][i])
        for g in range(CH // 16):
            idxv = idx_d[pl.ds(g * 16, 16)]
            plsc.store_scatter(postab, [idxv], g * 16 + lanes)
        totv = jnp.zeros((16,), jnp.int32)
        for g in range(CH // 16):
            idxv = idx_d[pl.ds(g * 16, 16)]
            posv = g * 16 + lanes
            winv = plsc.load_gather(postab, [idxv])
            loserv = winv != posv
            totv = totv + plsc.all_reduce_population_count(loserv)
            winbuf[pl.ds(g * 16, 16)] = winv
            lbuf[pl.ds(g * 16, 16)] = jnp.where(loserv, 1.0, 0.0)
            idx_d[pl.ds(g * 16, 16)] = jnp.where(loserv, N, idxv)

        @pl.when(totv[0] > 0)
        def _merge():
            for g in range(CH // 16):
                winv = winbuf[pl.ds(g * 16, 16)]
                lfv = lbuf[pl.ds(g * 16, 16)]
                for i in range(16):
                    e = g * 16 + i
                    w = winv[i]
                    lf = lfv[i]
                    for k in range(ACCW // 16):
                        cs = pl.ds(k * 16, 16)
                        acc_w = srow[w, cs] + srow[e, cs] * lf
                        srow[w, cs] = acc_w

        pltpu.sync_copy(srow, acc.at[idx_d], add=True)
        return 0

    lax.fori_loop(0, NCHUNK, _chunk, 0)
    plsc.subcore_barrier()

    pltpu.sync_copy(acc.at[pl.ds(row0, ROWS_PER_TILE), pl.ds(0, 128)],
                    num_hbm.at[cid, pl.ds(row0, ROWS_PER_TILE)])
    pltpu.sync_copy(acc.at[pl.ds(row0, ROWS_PER_TILE), pl.ds(128, 16)],
                    den_hbm.at[cid, pl.ds(row0, ROWS_PER_TILE)])


_edge_kernel = functools.partial(
    pl.kernel,
    out_type=[
        jax.ShapeDtypeStruct((NC, N, 128), jnp.float32),
        jax.ShapeDtypeStruct((NC, N, 16), jnp.float32),
    ],
    mesh=plsc.VectorSubcoreMesh(core_axis_name="c", subcore_axis_name="s",
                                num_cores=NC, num_subcores=NS),
    scratch_types=[
        pltpu.VMEM_SHARED((ACCN, ACCW), jnp.float32),
        pltpu.VMEM((CH,), jnp.int32),
        pltpu.VMEM((CH,), jnp.int32),
        pltpu.VMEM((CH,), jnp.float32),
        pltpu.VMEM((CH, 16), jnp.float32),
        pltpu.VMEM((CH, 16), jnp.float32),
        pltpu.VMEM((CH, 128), jnp.float32),
        pltpu.VMEM((CH, ACCW), jnp.float32),
        pltpu.VMEM((HEAD, CH), jnp.float32),
        pltpu.VMEM((N,), jnp.int32),
        pltpu.VMEM((CH,), jnp.int32),
        pltpu.VMEM((CH,), jnp.float32),
        pltpu.SemaphoreType.DMA,
    ],
    compiler_params=pltpu.CompilerParams(use_tc_tiling_on_sc=False,
                                          needs_layout_passes=False),
)(_edge_body)


def _combine_body(num_ref, den_ref, k_ref, out_ref):
    nsum = num_ref[0] + num_ref[1]
    dsum = den_ref[0] + den_ref[1]
    r = 1.0 / jnp.maximum(dsum[:, :4], 1e-38)
    out_ref[...] = nsum * jnp.dot(r, k_ref[...],
                                  preferred_element_type=jnp.float32)


def _combine(num, den, kmat):
    return pl.pallas_call(
        _combine_body,
        grid=(N // 200,),
        in_specs=[
            pl.BlockSpec((2, 200, 128), lambda i: (0, i, 0)),
            pl.BlockSpec((2, 200, 16), lambda i: (0, i, 0)),
            pl.BlockSpec((4, 128), lambda i: (0, 0)),
        ],
        out_specs=pl.BlockSpec((200, 128), lambda i: (i, 0)),
        out_shape=jax.ShapeDtypeStruct((N, 128), jnp.float32),
    )(num, den, kmat)


def kernel(x, edge_index, d_w, d_b, f_w, f_b, W, b_w, a_w, a_b):
    src = edge_index[0]
    dst = edge_index[1]
    xd = x @ d_w + d_b
    s = xd[src]
    d = xd[dst]
    score = jnp.tanh(jnp.concatenate([s, d, s - d], axis=-1) @ f_w + f_b)
    sgn = jnp.sign(score)[:, 0]

    a1 = a_w[:HD, 0]
    a2 = a_w[HD:, 0]
    eye4 = jnp.eye(HEAD, dtype=jnp.float32)
    P = jnp.kron(eye4, a1[:, None])
    Q = jnp.kron(eye4, a2[:, None])
    g = jnp.zeros((128, 16), jnp.float32)
    g = g.at[:, 0:4].set(P)
    g = g.at[:, 4:8].set(Q)
    tb = jnp.zeros((1, 16), jnp.float32)
    tb = tb.at[0, 4:8].set(a_b[0])
    kmat = jnp.kron(eye4, jnp.ones((1, HD), jnp.float32))

    xp = jnp.pad(x, ((0, NP - N), (0, 0)))
    h, t = _prep(xp, W, b_w[None, :], g, tb)

    num, den = _edge_kernel(src, dst, sgn, t, h)
    return _combine(num, den, kmat)


# speedup vs baseline: 17.8724x; 1.0948x over previous
import functools
import jax
import jax.numpy as jnp
from jax import lax
from jax.experimental import pallas as pl
from jax.experimental.pallas import tpu as pltpu
from jax.experimental.pallas import tpu_sc as plsc

N = 10000
E = 320000
D = 128
HEAD = 4
HD = 32
NP = 10240
NC = 2
NS = 16
NW = NC * NS
EPW = E // NW
CH = 80
NCHUNK = EPW // CH
ROWS_PER_TILE = N // NS
ACCN = N + 8


def _prep_body(x_ref, w_ref, bw_ref, g_ref, tb_ref, h_ref, t_ref):
    xb = x_ref[...].astype(jnp.bfloat16)
    wb = w_ref[...].astype(jnp.bfloat16)
    h = jnp.dot(xb, wb, preferred_element_type=jnp.float32) + bw_ref[...]
    h_ref[...] = h
    hb = h.astype(jnp.bfloat16)
    gb = g_ref[...].astype(jnp.bfloat16)
    t_ref[...] = jnp.dot(hb, gb,
                         preferred_element_type=jnp.float32) + tb_ref[...]


def _prep(xp, w, bw, g, tb):
    return pl.pallas_call(
        _prep_body,
        grid=(NP // 256,),
        in_specs=[
            pl.BlockSpec((256, 128), lambda i: (i, 0)),
            pl.BlockSpec((128, 128), lambda i: (0, 0)),
            pl.BlockSpec((1, 128), lambda i: (0, 0)),
            pl.BlockSpec((128, 16), lambda i: (0, 0)),
            pl.BlockSpec((1, 16), lambda i: (0, 0)),
        ],
        out_specs=[
            pl.BlockSpec((256, 128), lambda i: (i, 0)),
            pl.BlockSpec((256, 16), lambda i: (i, 0)),
        ],
        out_shape=[
            jax.ShapeDtypeStruct((NP, 128), jnp.float32),
            jax.ShapeDtypeStruct((NP, 16), jnp.float32),
        ],
    )(xp, w, bw, g, tb)


def _edge_body(src_hbm, dst_hbm, sgn_hbm, t_hbm, h_hbm, num_hbm, den_hbm,
               accn, accd, exb, wbuf, postab, winbuf, lbuf,
               idx_sA, idx_dA, sgA, bufsA, bufdA, bufhA, semA,
               idx_sB, idx_dB, sgB, bufsB, bufdB, bufhB, semB):
    cid = lax.axis_index("c")
    sid = lax.axis_index("s")
    wid = cid * NS + sid
    zero16 = jnp.zeros((16,), jnp.float32)

    def _zrow(r, _):
        for j in range(8):
            bufhA[r, pl.ds(j * 16, 16)] = zero16
        exb[r, pl.ds(0, 16)] = zero16
        return 0
    lax.fori_loop(0, CH, _zrow, 0)
    row0 = sid * ROWS_PER_TILE
    for tch in range(ROWS_PER_TILE // CH):
        pltpu.sync_copy(bufhA, accn.at[pl.ds(row0 + tch * CH, CH)])
        pltpu.sync_copy(exb, accd.at[pl.ds(row0 + tch * CH, CH)])
    rem = ROWS_PER_TILE % CH
    if rem:
        base = row0 + (ROWS_PER_TILE // CH) * CH
        pltpu.sync_copy(bufhA.at[pl.ds(0, rem)], accn.at[pl.ds(base, rem)])
        pltpu.sync_copy(exb.at[pl.ds(0, rem)], accd.at[pl.ds(base, rem)])
    @pl.when(sid == 0)
    def _zero_spare():
        pltpu.sync_copy(bufhA.at[pl.ds(0, ACCN - N)], accn.at[pl.ds(N, ACCN - N)])
        pltpu.sync_copy(exb.at[pl.ds(0, ACCN - N)], accd.at[pl.ds(N, ACCN - N)])
    plsc.subcore_barrier()

    lanes = lax.iota(jnp.int32, 16)
    ebase = wid * EPW

    def _load_idx(c, idx_s, idx_d, sg):
        off = ebase + c * CH
        pltpu.sync_copy(src_hbm.at[pl.ds(off, CH)], idx_s)
        pltpu.sync_copy(dst_hbm.at[pl.ds(off, CH)], idx_d)
        pltpu.sync_copy(sgn_hbm.at[pl.ds(off, CH)], sg)

    def _issue(idx_s, idx_d, bufs, bufd, bufh, sem):
        pltpu.async_copy(t_hbm.at[idx_s], bufs, sem)
        pltpu.async_copy(t_hbm.at[idx_d], bufd, sem)
        pltpu.async_copy(h_hbm.at[idx_s], bufh, sem)

    def _drain(idx_s, idx_d, bufs, bufd, bufh, sem):
        pltpu.make_async_copy(t_hbm.at[idx_s], bufs, sem).wait()
        pltpu.make_async_copy(t_hbm.at[idx_d], bufd, sem).wait()
        pltpu.make_async_copy(h_hbm.at[idx_s], bufh, sem).wait()

    def _process(idx_d, sg, bufs, bufd, bufh):
        for g in range(CH // 16):
            rows = g * 16 + lanes
            col = lambda j: jnp.full((16,), j, jnp.int32)
            sgn = sg[pl.ds(g * 16, 16)]
            for hh in range(HEAD):
                ps = plsc.load_gather(bufs, [rows, col(hh)])
                qd = plsc.load_gather(bufd, [rows, col(4 + hh)])
                t = sgn * ps + qd
                al = jnp.where(t >= 0.0, t, 0.01 * t)
                ex = jnp.exp(al)
                plsc.store_scatter(exb, [rows, col(hh)], ex)
                wbuf[hh, pl.ds(g * 16, 16)] = ex * sgn

        def _scaleg(g, _):
            wv = [wbuf[hh, pl.ds(g * 16, 16)] for hh in range(HEAD)]
            for i in range(16):
                e = g * 16 + i
                for k in range(8):
                    cs = pl.ds(k * 16, 16)
                    bufh[e, cs] = bufh[e, cs] * wv[k // 2][i]
            return 0
        lax.fori_loop(0, CH // 16, _scaleg, 0)

        for g in range(CH // 16):
            idxv = idx_d[pl.ds(g * 16, 16)]
            plsc.store_scatter(postab, [idxv], g * 16 + lanes)
        totv = jnp.zeros((16,), jnp.int32)
        for g in range(CH // 16):
            idxv = idx_d[pl.ds(g * 16, 16)]
            posv = g * 16 + lanes
            winv = plsc.load_gather(postab, [idxv])
            loserv = winv != posv
            totv = totv + plsc.all_reduce_population_count(loserv)
            winbuf[pl.ds(g * 16, 16)] = winv
            lbuf[pl.ds(g * 16, 16)] = jnp.where(loserv, 1.0, 0.0)
            idx_d[pl.ds(g * 16, 16)] = jnp.where(loserv, N, idxv)

        @pl.when(totv[0] > 0)
        def _merge():
            def _mergeg(g, _):
                winv = winbuf[pl.ds(g * 16, 16)]
                lfv = lbuf[pl.ds(g * 16, 16)]
                for i in range(16):
                    e = g * 16 + i
                    w = winv[i]
                    lf = lfv[i]
                    for k in range(8):
                        cs = pl.ds(k * 16, 16)
                        bufh[w, cs] = bufh[w, cs] + bufh[e, cs] * lf
                    cs = pl.ds(0, 16)
                    exb[w, cs] = exb[w, cs] + exb[e, cs] * lf
                return 0
            lax.fori_loop(0, CH // 16, _mergeg, 0)

        pltpu.sync_copy(bufh, accn.at[idx_d], add=True)
        pltpu.sync_copy(exb, accd.at[idx_d], add=True)

    A = (idx_sA, idx_dA, sgA, bufsA, bufdA, bufhA, semA)
    B = (idx_sB, idx_dB, sgB, bufsB, bufdB, bufhB, semB)

    def _proc_set(s):
        _process(s[1], s[2], s[3], s[4], s[5])

    _load_idx(0, A[0], A[1], A[2])
    _issue(A[0], A[1], A[3], A[4], A[5], A[6])

    def _pair(c2, _):
        _drain(A[0], A[1], A[3], A[4], A[5], A[6])
        _load_idx(2 * c2 + 1, B[0], B[1], B[2])
        _issue(B[0], B[1], B[3], B[4], B[5], B[6])
        _proc_set(A)
        _drain(B[0], B[1], B[3], B[4], B[5], B[6])
        _load_idx(2 * c2 + 2, A[0], A[1], A[2])
        _issue(A[0], A[1], A[3], A[4], A[5], A[6])
        _proc_set(B)
        return 0

    lax.fori_loop(0, NCHUNK // 2, _pair, 0)
    _drain(A[0], A[1], A[3], A[4], A[5], A[6])
    _proc_set(A)
    plsc.subcore_barrier()

    pltpu.sync_copy(accn.at[pl.ds(row0, ROWS_PER_TILE)],
                    num_hbm.at[cid, pl.ds(row0, ROWS_PER_TILE)])
    pltpu.sync_copy(accd.at[pl.ds(row0, ROWS_PER_TILE)],
                    den_hbm.at[cid, pl.ds(row0, ROWS_PER_TILE)])


_edge_kernel = functools.partial(
    pl.kernel,
    out_type=[
        jax.ShapeDtypeStruct((NC, N, 128), jnp.float32),
        jax.ShapeDtypeStruct((NC, N, 16), jnp.float32),
    ],
    mesh=plsc.VectorSubcoreMesh(core_axis_name="c", subcore_axis_name="s",
                                num_cores=NC, num_subcores=NS),
    scratch_types=(
        [
            pltpu.VMEM_SHARED((ACCN, 128), jnp.float32),
            pltpu.VMEM_SHARED((ACCN, 16), jnp.float32),
            pltpu.VMEM((CH, 16), jnp.float32),
            pltpu.VMEM((HEAD, CH), jnp.float32),
            pltpu.VMEM((N,), jnp.int32),
            pltpu.VMEM((CH,), jnp.int32),
            pltpu.VMEM((CH,), jnp.float32),
        ]
        + 2 * [
            pltpu.VMEM((CH,), jnp.int32),
            pltpu.VMEM((CH,), jnp.int32),
            pltpu.VMEM((CH,), jnp.float32),
            pltpu.VMEM((CH, 16), jnp.float32),
            pltpu.VMEM((CH, 16), jnp.float32),
            pltpu.VMEM((CH, 128), jnp.float32),
            pltpu.SemaphoreType.DMA,
        ]
    ),
    compiler_params=pltpu.CompilerParams(use_tc_tiling_on_sc=False,
                                          needs_layout_passes=False),
)(_edge_body)


def _combine_body(num_ref, den_ref, k_ref, out_ref):
    nsum = num_ref[0] + num_ref[1]
    dsum = den_ref[0] + den_ref[1]
    r = 1.0 / jnp.maximum(dsum[:, :4], 1e-38)
    out_ref[...] = nsum * jnp.dot(r, k_ref[...],
                                  preferred_element_type=jnp.float32)


def _combine(num, den, kmat):
    return pl.pallas_call(
        _combine_body,
        grid=(N // 200,),
        in_specs=[
            pl.BlockSpec((2, 200, 128), lambda i: (0, i, 0)),
            pl.BlockSpec((2, 200, 16), lambda i: (0, i, 0)),
            pl.BlockSpec((4, 128), lambda i: (0, 0)),
        ],
        out_specs=pl.BlockSpec((200, 128), lambda i: (i, 0)),
        out_shape=jax.ShapeDtypeStruct((N, 128), jnp.float32),
    )(num, den, kmat)


def kernel(x, edge_index, d_w, d_b, f_w, f_b, W, b_w, a_w, a_b):
    src = edge_index[0]
    dst = edge_index[1]
    xd = x @ d_w + d_b
    s = xd[src]
    d = xd[dst]
    score = jnp.tanh(jnp.concatenate([s, d, s - d], axis=-1) @ f_w + f_b)
    sgn = jnp.sign(score)[:, 0]

    a1 = a_w[:HD, 0]
    a2 = a_w[HD:, 0]
    eye4 = jnp.eye(HEAD, dtype=jnp.float32)
    P = jnp.kron(eye4, a1[:, None])
    Q = jnp.kron(eye4, a2[:, None])
    g = jnp.zeros((128, 16), jnp.float32)
    g = g.at[:, 0:4].set(P)
    g = g.at[:, 4:8].set(Q)
    tb = jnp.zeros((1, 16), jnp.float32)
    tb = tb.at[0, 4:8].set(a_b[0])
    kmat = jnp.kron(eye4, jnp.ones((1, HD), jnp.float32))

    xp = jnp.pad(x, ((0, NP - N), (0, 0)))
    h, t = _prep(xp, W, b_w[None, :], g, tb)

    num, den = _edge_kernel(src, dst, sgn, t, h)
    return _combine(num, den, kmat)


# speedup vs baseline: 17.8771x; 1.0003x over previous
import functools
import jax
import jax.numpy as jnp
from jax import lax
from jax.experimental import pallas as pl
from jax.experimental.pallas import tpu as pltpu
from jax.experimental.pallas import tpu_sc as plsc

N = 10000
E = 320000
D = 128
HEAD = 4
HD = 32
NP = 10240
NC = 2
NS = 16
NW = NC * NS
EPW = E // NW
CH = 80
NCHUNK = EPW // CH
ROWS_PER_TILE = N // NS
ACCN = N + 8


def _prep_body(x_ref, w_ref, bw_ref, g_ref, tb_ref, h_ref, t_ref):
    xb = x_ref[...].astype(jnp.bfloat16)
    wb = w_ref[...].astype(jnp.bfloat16)
    h = jnp.dot(xb, wb, preferred_element_type=jnp.float32) + bw_ref[...]
    h_ref[...] = h
    hb = h.astype(jnp.bfloat16)
    gb = g_ref[...].astype(jnp.bfloat16)
    t_ref[...] = jnp.dot(hb, gb,
                         preferred_element_type=jnp.float32) + tb_ref[...]


def _prep(xp, w, bw, g, tb):
    return pl.pallas_call(
        _prep_body,
        grid=(NP // 256,),
        in_specs=[
            pl.BlockSpec((256, 128), lambda i: (i, 0)),
            pl.BlockSpec((128, 128), lambda i: (0, 0)),
            pl.BlockSpec((1, 128), lambda i: (0, 0)),
            pl.BlockSpec((128, 16), lambda i: (0, 0)),
            pl.BlockSpec((1, 16), lambda i: (0, 0)),
        ],
        out_specs=[
            pl.BlockSpec((256, 128), lambda i: (i, 0)),
            pl.BlockSpec((256, 16), lambda i: (i, 0)),
        ],
        out_shape=[
            jax.ShapeDtypeStruct((NP, 128), jnp.float32),
            jax.ShapeDtypeStruct((NP, 16), jnp.float32),
        ],
    )(xp, w, bw, g, tb)


def _edge_body(src_hbm, dst_hbm, sgn_hbm, t_hbm, h_hbm, num_hbm, den_hbm,
               accn, accd, exb, wbuf, postab, winbuf, lbuf,
               idx_sA, idx_dA, sgA, bufsA, bufdA, bufhA, semA,
               idx_sB, idx_dB, sgB, bufsB, bufdB, bufhB, semB):
    cid = lax.axis_index("c")
    sid = lax.axis_index("s")
    wid = cid * NS + sid
    zero16 = jnp.zeros((16,), jnp.float32)

    def _zrow(r, _):
        for j in range(8):
            bufhA[r, pl.ds(j * 16, 16)] = zero16
        exb[r, pl.ds(0, 16)] = zero16
        return 0
    lax.fori_loop(0, CH, _zrow, 0)
    row0 = sid * ROWS_PER_TILE
    for tch in range(ROWS_PER_TILE // CH):
        pltpu.sync_copy(bufhA, accn.at[pl.ds(row0 + tch * CH, CH)])
        pltpu.sync_copy(exb, accd.at[pl.ds(row0 + tch * CH, CH)])
    rem = ROWS_PER_TILE % CH
    if rem:
        base = row0 + (ROWS_PER_TILE // CH) * CH
        pltpu.sync_copy(bufhA.at[pl.ds(0, rem)], accn.at[pl.ds(base, rem)])
        pltpu.sync_copy(exb.at[pl.ds(0, rem)], accd.at[pl.ds(base, rem)])
    @pl.when(sid == 0)
    def _zero_spare():
        pltpu.sync_copy(bufhA.at[pl.ds(0, ACCN - N)], accn.at[pl.ds(N, ACCN - N)])
        pltpu.sync_copy(exb.at[pl.ds(0, ACCN - N)], accd.at[pl.ds(N, ACCN - N)])
    plsc.subcore_barrier()

    lanes = lax.iota(jnp.int32, 16)
    ebase = wid * EPW

    def _load_idx(c, idx_s, idx_d, sg, sem):
        del sem
        off = ebase + c * CH
        pltpu.sync_copy(src_hbm.at[pl.ds(off, CH)], idx_s)
        pltpu.sync_copy(dst_hbm.at[pl.ds(off, CH)], idx_d)
        pltpu.sync_copy(sgn_hbm.at[pl.ds(off, CH)], sg)

    def _issue(idx_s, idx_d, bufs, bufd, bufh, sem):
        pltpu.async_copy(t_hbm.at[idx_s], bufs, sem)
        pltpu.async_copy(t_hbm.at[idx_d], bufd, sem)
        pltpu.async_copy(h_hbm.at[idx_s], bufh, sem)

    def _drain(idx_s, idx_d, bufs, bufd, bufh, sem):
        pltpu.make_async_copy(t_hbm.at[idx_s], bufs, sem).wait()
        pltpu.make_async_copy(t_hbm.at[idx_d], bufd, sem).wait()
        pltpu.make_async_copy(h_hbm.at[idx_s], bufh, sem).wait()

    def _process(idx_d, sg, bufs, bufd, bufh, sem):
        for g in range(CH // 16):
            rows = g * 16 + lanes
            col = lambda j: jnp.full((16,), j, jnp.int32)
            sgn = sg[pl.ds(g * 16, 16)]
            for hh in range(HEAD):
                ps = plsc.load_gather(bufs, [rows, col(hh)])
                qd = plsc.load_gather(bufd, [rows, col(4 + hh)])
                t = sgn * ps + qd
                al = jnp.where(t >= 0.0, t, 0.01 * t)
                ex = jnp.exp(al)
                plsc.store_scatter(exb, [rows, col(hh)], ex)
                wbuf[hh, pl.ds(g * 16, 16)] = ex * sgn

        def _scaleg(g, _):
            wv = [wbuf[hh, pl.ds(g * 16, 16)] for hh in range(HEAD)]
            for i in range(16):
                e = g * 16 + i
                for k in range(8):
                    cs = pl.ds(k * 16, 16)
                    bufh[e, cs] = bufh[e, cs] * wv[k // 2][i]
            return 0
        lax.fori_loop(0, CH // 16, _scaleg, 0)

        for g in range(CH // 16):
            idxv = idx_d[pl.ds(g * 16, 16)]
            plsc.store_scatter(postab, [idxv], g * 16 + lanes)
        totv = jnp.zeros((16,), jnp.int32)
        for g in range(CH // 16):
            idxv = idx_d[pl.ds(g * 16, 16)]
            posv = g * 16 + lanes
            winv = plsc.load_gather(postab, [idxv])
            loserv = winv != posv
            totv = totv + plsc.all_reduce_population_count(loserv)
            winbuf[pl.ds(g * 16, 16)] = winv
            lbuf[pl.ds(g * 16, 16)] = jnp.where(loserv, 1.0, 0.0)
            idx_d[pl.ds(g * 16, 16)] = jnp.where(loserv, N, idxv)

        @pl.when(totv[0] > 0)
        def _merge():
            def _mergeg(g, _):
                winv = winbuf[pl.ds(g * 16, 16)]
                lfv = lbuf[pl.ds(g * 16, 16)]
                for i in range(16):
                    e = g * 16 + i
                    w = winv[i]
                    lf = lfv[i]
                    for k in range(8):
                        cs = pl.ds(k * 16, 16)
                        bufh[w, cs] = bufh[w, cs] + bufh[e, cs] * lf
                    cs = pl.ds(0, 16)
                    exb[w, cs] = exb[w, cs] + exb[e, cs] * lf
                return 0
            lax.fori_loop(0, CH // 16, _mergeg, 0)

        pltpu.sync_copy(bufh, accn.at[idx_d], add=True)
        pltpu.sync_copy(exb, accd.at[idx_d], add=True)

    A = (idx_sA, idx_dA, sgA, bufsA, bufdA, bufhA, semA)
    B = (idx_sB, idx_dB, sgB, bufsB, bufdB, bufhB, semB)

    def _proc_set(s):
        _process(s[1], s[2], s[3], s[4], s[5], s[6])

    _load_idx(0, A[0], A[1], A[2], A[6])
    _issue(A[0], A[1], A[3], A[4], A[5], A[6])

    def _pair(c2, _):
        _drain(A[0], A[1], A[3], A[4], A[5], A[6])
        _load_idx(2 * c2 + 1, B[0], B[1], B[2], B[6])
        _issue(B[0], B[1], B[3], B[4], B[5], B[6])
        _proc_set(A)
        _drain(B[0], B[1], B[3], B[4], B[5], B[6])
        _load_idx(2 * c2 + 2, A[0], A[1], A[2], A[6])
        _issue(A[0], A[1], A[3], A[4], A[5], A[6])
        _proc_set(B)
        return 0

    lax.fori_loop(0, NCHUNK // 2, _pair, 0)
    _drain(A[0], A[1], A[3], A[4], A[5], A[6])
    _proc_set(A)
    plsc.subcore_barrier()

    pltpu.sync_copy(accn.at[pl.ds(row0, ROWS_PER_TILE)],
                    num_hbm.at[cid, pl.ds(row0, ROWS_PER_TILE)])
    pltpu.sync_copy(accd.at[pl.ds(row0, ROWS_PER_TILE)],
                    den_hbm.at[cid, pl.ds(row0, ROWS_PER_TILE)])


_edge_kernel = functools.partial(
    pl.kernel,
    out_type=[
        jax.ShapeDtypeStruct((NC, N, 128), jnp.float32),
        jax.ShapeDtypeStruct((NC, N, 16), jnp.float32),
    ],
    mesh=plsc.VectorSubcoreMesh(core_axis_name="c", subcore_axis_name="s",
                                num_cores=NC, num_subcores=NS),
    scratch_types=(
        [
            pltpu.VMEM_SHARED((ACCN, 128), jnp.float32),
            pltpu.VMEM_SHARED((ACCN, 16), jnp.float32),
            pltpu.VMEM((CH, 16), jnp.float32),
            pltpu.VMEM((HEAD, CH), jnp.float32),
            pltpu.VMEM((N,), jnp.int32),
            pltpu.VMEM((CH,), jnp.int32),
            pltpu.VMEM((CH,), jnp.float32),
        ]
        + 2 * [
            pltpu.VMEM((CH,), jnp.int32),
            pltpu.VMEM((CH,), jnp.int32),
            pltpu.VMEM((CH,), jnp.float32),
            pltpu.VMEM((CH, 16), jnp.float32),
            pltpu.VMEM((CH, 16), jnp.float32),
            pltpu.VMEM((CH, 128), jnp.float32),
            pltpu.SemaphoreType.DMA,
        ]
    ),
    compiler_params=pltpu.CompilerParams(use_tc_tiling_on_sc=False,
                                          needs_layout_passes=False),
)(_edge_body)


def _combine_body(num_ref, den_ref, k_ref, out_ref):
    nsum = num_ref[0] + num_ref[1]
    dsum = den_ref[0] + den_ref[1]
    r = 1.0 / jnp.maximum(dsum[:, :4], 1e-38)
    out_ref[...] = nsum * jnp.dot(r, k_ref[...],
                                  preferred_element_type=jnp.float32)


def _combine(num, den, kmat):
    return pl.pallas_call(
        _combine_body,
        grid=(N // 200,),
        in_specs=[
            pl.BlockSpec((2, 200, 128), lambda i: (0, i, 0)),
            pl.BlockSpec((2, 200, 16), lambda i: (0, i, 0)),
            pl.BlockSpec((4, 128), lambda i: (0, 0)),
        ],
        out_specs=pl.BlockSpec((200, 128), lambda i: (i, 0)),
        out_shape=jax.ShapeDtypeStruct((N, 128), jnp.float32),
    )(num, den, kmat)


def kernel(x, edge_index, d_w, d_b, f_w, f_b, W, b_w, a_w, a_b):
    src = edge_index[0]
    dst = edge_index[1]
    xd = x @ d_w + d_b
    s = xd[src]
    d = xd[dst]
    score = jnp.tanh(jnp.concatenate([s, d, s - d], axis=-1) @ f_w + f_b)
    sgn = jnp.sign(score)[:, 0]

    a1 = a_w[:HD, 0]
    a2 = a_w[HD:, 0]
    eye4 = jnp.eye(HEAD, dtype=jnp.float32)
    P = jnp.kron(eye4, a1[:, None])
    Q = jnp.kron(eye4, a2[:, None])
    g = jnp.zeros((128, 16), jnp.float32)
    g = g.at[:, 0:4].set(P)
    g = g.at[:, 4:8].set(Q)
    tb = jnp.zeros((1, 16), jnp.float32)
    tb = tb.at[0, 4:8].set(a_b[0])
    kmat = jnp.kron(eye4, jnp.ones((1, HD), jnp.float32))

    xp = jnp.pad(x, ((0, NP - N), (0, 0)))
    h, t = _prep(xp, W, b_w[None, :], g, tb)

    num, den = _edge_kernel(src, dst, sgn, t, h)
    return _combine(num, den, kmat)
